# baseline (speedup 1.0000x reference)
import functools

import jax
import jax.numpy as jnp
from jax.experimental import pallas as pl

_TILE = 10000


def _layer(h, w_ref, a, relu):
    t = jax.lax.dot_general(h, w_ref[...], (((1,), (1,)), ((), ())),
                            preferred_element_type=jnp.float32)
    t = jnp.dot(t, a, preferred_element_type=jnp.float32)
    return jnp.maximum(t, 0.0) if relu else t


def _gcn_body(h_ref, a_ref, win_ref, w1_ref, w2_ref, wout_ref, o_ref):
    a = a_ref[...]
    h = h_ref[...]
    h = _layer(h, win_ref, a, True)
    h = _layer(h, w1_ref, a, True)
    h = _layer(h, w2_ref, a, True)
    m = jax.lax.dot_general(wout_ref[...], a, (((0,), (0,)), ((), ())),
                            preferred_element_type=jnp.float32)
    o_ref[...] = jnp.dot(h, m, preferred_element_type=jnp.float32)


@functools.partial(jax.jit, static_argnames=())
def kernel(x, adjacency_hat, W_in, W_h1, W_h2, W_out):
    B, N, D = x.shape
    h = x.reshape(B * N, D)
    rows = B * N
    grid = (rows // _TILE,)
    wspec = pl.BlockSpec((D, D), lambda i: (0, 0))
    out = pl.pallas_call(
        _gcn_body,
        grid=grid,
        in_specs=[
            pl.BlockSpec((_TILE, D), lambda i: (i, 0)),
            wspec, wspec, wspec, wspec, wspec,
        ],
        out_specs=pl.BlockSpec((_TILE, D), lambda i: (i, 0)),
        out_shape=jax.ShapeDtypeStruct((rows, D), jnp.float32),
    )(h, adjacency_hat, W_in, W_h1, W_h2, W_out)
    return out


# speedup vs baseline: 1.1271x; 1.1271x over previous
import functools

import jax
import jax.numpy as jnp
from jax.experimental import pallas as pl
from jax.experimental.pallas import tpu as pltpu

_TILE = 10000


def _layer(h, w_ref, a, relu):
    t = jax.lax.dot_general(h, w_ref[...], (((1,), (1,)), ((), ())),
                            preferred_element_type=jnp.float32)
    t = jnp.dot(t, a, preferred_element_type=jnp.float32)
    return jnp.maximum(t, 0.0) if relu else t


def _gcn_body(h_ref, a_ref, win_ref, w1_ref, w2_ref, wout_ref, o_ref, m_ref):
    @pl.when(pl.program_id(0) == 0)
    def _fold():
        m_ref[...] = jax.lax.dot_general(
            wout_ref[...], a_ref[...], (((0,), (0,)), ((), ())),
            preferred_element_type=jnp.float32)

    a = a_ref[...]
    h = h_ref[...]
    h = _layer(h, win_ref, a, True)
    h = _layer(h, w1_ref, a, True)
    h = _layer(h, w2_ref, a, True)
    o_ref[...] = jnp.dot(h, m_ref[...], preferred_element_type=jnp.float32)


@functools.partial(jax.jit, static_argnames=())
def kernel(x, adjacency_hat, W_in, W_h1, W_h2, W_out):
    B, N, D = x.shape
    h = x.reshape(B * N, D)
    rows = B * N
    grid = (rows // _TILE,)
    wspec = pl.BlockSpec((D, D), lambda i: (0, 0))
    out = pl.pallas_call(
        _gcn_body,
        grid=grid,
        in_specs=[
            pl.BlockSpec((_TILE, D), lambda i: (i, 0)),
            wspec, wspec, wspec, wspec, wspec,
        ],
        out_specs=pl.BlockSpec((_TILE, D), lambda i: (i, 0)),
        out_shape=jax.ShapeDtypeStruct((rows, D), jnp.float32),
        scratch_shapes=[pltpu.VMEM((D, D), jnp.float32)],
    )(h, adjacency_hat, W_in, W_h1, W_h2, W_out)
    return out
